# baseline (device time: 21132 ns/iter reference)
import jax
import jax.numpy as jnp
from jax import lax
from jax.experimental import pallas as pl
from jax.experimental.pallas import tpu as pltpu

N_DEV = 8
N_EXP_LOCAL = 4
N_EXP = 32


def kernel(x, router_W, route_idx, expert_W, shared_W):
    n, d = x.shape
    h = shared_W.shape[1]
    chunk = n // N_DEV
    half = n // 2
    quarter = n // 4

    def body(x_ref, rw_ref, idx_ref, ew_ref, sw_ref, out_ref,
             gate_ref, acc_ref, stg1, stg2, stg3, rcv1, rcv2, rcv3,
             send_sems, recv_sems):
        my = lax.axis_index("i")
        quad = my // 4
        r = lax.rem(my, 4)
        pq = r // 2
        b0 = lax.rem(r, 2)
        pz = lax.rem(my + 4, N_DEV)
        py = 4 * quad + 3 - r
        px = 4 * quad + 2 * pq + (1 - b0)

        barrier_sem = pltpu.get_barrier_semaphore()
        for nbr in (pz, py, px):
            pl.semaphore_signal(barrier_sem, inc=1, device_id=(nbr,),
                                device_id_type=pl.DeviceIdType.MESH)
        pl.semaphore_wait(barrier_sem, 3)

        xv = x_ref[:, :]
        scores = jnp.dot(xv, rw_ref[:, :], preferred_element_type=jnp.float32)
        m = jnp.max(scores, axis=1, keepdims=True)
        p = jnp.exp(scores - m)
        probs = p / jnp.sum(p, axis=1, keepdims=True)
        idx = idx_ref[:, :]
        eids = lax.broadcasted_iota(jnp.int32, (n, N_EXP), 1)
        gate_ref[:, :] = jnp.sum(jnp.where(eids == idx, probs, 0.0), axis=1,
                                 keepdims=True)

        ewb = jnp.reshape(ew_ref[:, :, :],
                          (N_EXP_LOCAL * d, h)).astype(jnp.bfloat16)

        def partial_rows(row0, rows):
            sl = pl.ds(row0, rows)
            xs = x_ref[sl, :]
            idx_c = idx_ref[sl, :]
            gate_c = gate_ref[sl, :]
            parts = []
            for k in range(N_EXP_LOCAL):
                e = my * N_EXP_LOCAL + k
                w = jnp.where(idx_c == e, gate_c, 0.0)
                parts.append((xs * w).astype(jnp.bfloat16))
            xw = jnp.concatenate(parts, axis=1)
            return jnp.dot(xw, ewb, preferred_element_type=jnp.float32)

        def exchange(stg, rcv, peer, s):
            rdma = pltpu.make_async_remote_copy(
                src_ref=stg, dst_ref=rcv,
                send_sem=send_sems.at[s], recv_sem=recv_sems.at[s],
                device_id=(peer,), device_id_type=pl.DeviceIdType.MESH,
            )
            rdma.start()
            return rdma

        stg1[:, :] = partial_rows(half * (1 - quad), half).astype(jnp.bfloat16)
        rdma1 = exchange(stg1, rcv1, pz, 0)
        acc_ref[:, :] = partial_rows(half * quad, half)
        rdma1.wait_recv()
        acc_ref[:, :] = acc_ref[:, :] + rcv1[:, :].astype(jnp.float32)

        sent2 = pl.ds(quarter * (1 - pq), quarter)
        kept2 = pl.ds(quarter * pq, quarter)
        stg2[:, :] = acc_ref[sent2, :].astype(jnp.bfloat16)
        rdma2 = exchange(stg2, rcv2, py, 1)
        xs_my = x_ref[pl.ds(my * chunk, chunk), :]
        shared = jnp.dot(xs_my.astype(jnp.bfloat16),
                         sw_ref[:, :].astype(jnp.bfloat16),
                         preferred_element_type=jnp.float32)
        rdma2.wait_recv()
        acc_ref[kept2, :] = acc_ref[kept2, :] + rcv2[:, :].astype(jnp.float32)

        sent3 = pl.ds(chunk * (2 * pq + 1 - b0), chunk)
        stg3[:, :] = acc_ref[sent3, :].astype(jnp.bfloat16)
        rdma3 = exchange(stg3, rcv3, px, 2)
        rdma3.wait_recv()
        out_ref[:, :] = (shared + acc_ref[pl.ds(chunk * r, chunk), :]
                         + rcv3[:, :].astype(jnp.float32))

        rdma1.wait_send()
        rdma2.wait_send()
        rdma3.wait_send()

    return pl.pallas_call(
        body,
        out_shape=jax.ShapeDtypeStruct((chunk, h), jnp.float32),
        in_specs=[pl.BlockSpec(memory_space=pltpu.VMEM)] * 5,
        out_specs=pl.BlockSpec(memory_space=pltpu.VMEM),
        scratch_shapes=[
            pltpu.VMEM((n, 1), jnp.float32),
            pltpu.VMEM((half, h), jnp.float32),
            pltpu.VMEM((half, h), jnp.bfloat16),
            pltpu.VMEM((quarter, h), jnp.bfloat16),
            pltpu.VMEM((chunk, h), jnp.bfloat16),
            pltpu.VMEM((half, h), jnp.bfloat16),
            pltpu.VMEM((quarter, h), jnp.bfloat16),
            pltpu.VMEM((chunk, h), jnp.bfloat16),
            pltpu.SemaphoreType.DMA((3,)),
            pltpu.SemaphoreType.DMA((3,)),
        ],
        compiler_params=pltpu.CompilerParams(collective_id=0),
    )(x, router_W, route_idx, expert_W, shared_W)


# device time: 19048 ns/iter; 1.1094x vs baseline; 1.1094x over previous
import jax
import jax.numpy as jnp
from jax import lax
from jax.experimental import pallas as pl
from jax.experimental.pallas import tpu as pltpu

N_DEV = 8
N_EXP_LOCAL = 4
N_EXP = 32


def kernel(x, router_W, route_idx, expert_W, shared_W):
    n, d = x.shape
    h = shared_W.shape[1]
    chunk = n // N_DEV

    def body(x_ref, rw_ref, idx_ref, ew_ref, sw_ref, out_ref,
             gate_ref, ewb_ref, swb_ref, stga, rcva, stgb, rcvb,
             sems_a_send, sems_a_recv, sem_b_send, sem_b_recv):
        my = lax.axis_index("i")
        quad = my // 4
        r = lax.rem(my, 4)
        mirror = lax.rem(my + 4, N_DEV)

        barrier_sem = pltpu.get_barrier_semaphore()
        for o in range(1, 4):
            mate = 4 * quad + lax.rem(r + o, 4)
            pl.semaphore_signal(barrier_sem, inc=1, device_id=(mate,),
                                device_id_type=pl.DeviceIdType.MESH)
        pl.semaphore_signal(barrier_sem, inc=1, device_id=(mirror,),
                            device_id_type=pl.DeviceIdType.MESH)
        pl.semaphore_wait(barrier_sem, 4)

        xv = x_ref[:, :]
        scores = jnp.dot(xv, rw_ref[:, :], preferred_element_type=jnp.float32)
        m = jnp.max(scores, axis=1, keepdims=True)
        p = jnp.exp(scores - m)
        probs = p / jnp.sum(p, axis=1, keepdims=True)
        idx = idx_ref[:, :]
        eids = lax.broadcasted_iota(jnp.int32, (n, N_EXP), 1)
        gate_ref[:, :] = jnp.sum(jnp.where(eids == idx, probs, 0.0), axis=1,
                                 keepdims=True)

        for k in range(N_EXP_LOCAL):
            ewb_ref[k, :, :] = ew_ref[k, :, :].astype(jnp.bfloat16)
        swb_ref[:, :] = sw_ref[:, :].astype(jnp.bfloat16)

        def partial_chunk(dst):
            rows = pl.ds(dst * chunk, chunk)
            xs = x_ref[rows, :]
            idx_c = idx_ref[rows, :]
            gate_c = gate_ref[rows, :]
            acc = jnp.zeros((chunk, h), jnp.float32)
            for k in range(N_EXP_LOCAL):
                e = my * N_EXP_LOCAL + k
                w = jnp.where(idx_c == e, gate_c, 0.0)
                acc = acc + jnp.dot((xs * w).astype(jnp.bfloat16),
                                    ewb_ref[k, :, :],
                                    preferred_element_type=jnp.float32)
            return acc

        rdmas_a = []
        for o in range(1, 4):
            mate = 4 * quad + lax.rem(r + o, 4)
            mate_mirror_chunk = lax.rem(mate + 4, N_DEV)
            stga[o - 1, 0:chunk, :] = partial_chunk(mate).astype(jnp.bfloat16)
            stga[o - 1, chunk:2 * chunk, :] = (
                partial_chunk(mate_mirror_chunk).astype(jnp.bfloat16))
            rdma = pltpu.make_async_remote_copy(
                src_ref=stga.at[o - 1],
                dst_ref=rcva.at[o - 1],
                send_sem=sems_a_send.at[o - 1],
                recv_sem=sems_a_recv.at[o - 1],
                device_id=(mate,),
                device_id_type=pl.DeviceIdType.MESH,
            )
            rdma.start()
            rdmas_a.append(rdma)

        acc_own = partial_chunk(my)
        acc_mir = partial_chunk(mirror)
        xs_my = x_ref[pl.ds(my * chunk, chunk), :]
        shared = jnp.dot(xs_my.astype(jnp.bfloat16), swb_ref[:, :],
                         preferred_element_type=jnp.float32)

        for o in range(1, 4):
            rdmas_a[o - 1].wait_recv()
            acc_own = acc_own + rcva[o - 1, 0:chunk, :].astype(jnp.float32)
            acc_mir = acc_mir + rcva[o - 1, chunk:2 * chunk, :].astype(
                jnp.float32)

        stgb[:, :] = acc_mir.astype(jnp.bfloat16)
        rdma_b = pltpu.make_async_remote_copy(
            src_ref=stgb, dst_ref=rcvb,
            send_sem=sem_b_send, recv_sem=sem_b_recv,
            device_id=(mirror,), device_id_type=pl.DeviceIdType.MESH,
        )
        rdma_b.start()
        rdma_b.wait_recv()
        out_ref[:, :] = shared + acc_own + rcvb[:, :].astype(jnp.float32)

        for rd in rdmas_a:
            rd.wait_send()
        rdma_b.wait_send()

    return pl.pallas_call(
        body,
        out_shape=jax.ShapeDtypeStruct((chunk, h), jnp.float32),
        in_specs=[pl.BlockSpec(memory_space=pltpu.VMEM)] * 5,
        out_specs=pl.BlockSpec(memory_space=pltpu.VMEM),
        scratch_shapes=[
            pltpu.VMEM((n, 1), jnp.float32),
            pltpu.VMEM((N_EXP_LOCAL, d, h), jnp.bfloat16),
            pltpu.VMEM((d, h), jnp.bfloat16),
            pltpu.VMEM((3, 2 * chunk, h), jnp.bfloat16),
            pltpu.VMEM((3, 2 * chunk, h), jnp.bfloat16),
            pltpu.VMEM((chunk, h), jnp.bfloat16),
            pltpu.VMEM((chunk, h), jnp.bfloat16),
            pltpu.SemaphoreType.DMA((3,)),
            pltpu.SemaphoreType.DMA((3,)),
            pltpu.SemaphoreType.DMA,
            pltpu.SemaphoreType.DMA,
        ],
        compiler_params=pltpu.CompilerParams(collective_id=0),
    )(x, router_W, route_idx, expert_W, shared_W)


# device time: 17293 ns/iter; 1.2220x vs baseline; 1.1015x over previous
import jax
import jax.numpy as jnp
from jax import lax
from jax.experimental import pallas as pl
from jax.experimental.pallas import tpu as pltpu

N_DEV = 8
N_EXP_LOCAL = 4
N_EXP = 32


def kernel(x, router_W, route_idx, expert_W, shared_W):
    n, d = x.shape
    h = shared_W.shape[1]
    chunk = n // N_DEV

    def body(x_ref, rw_ref, idx_ref, ew_ref, sw_ref, out_ref,
             gate_ref, ewb_ref, swb_ref, send_ref, recv_ref,
             send_sems, recv_sems):
        my = lax.axis_index("i")

        barrier_sem = pltpu.get_barrier_semaphore()
        for t in range(1, N_DEV):
            peer = lax.rem(my + t, N_DEV)
            pl.semaphore_signal(barrier_sem, inc=1, device_id=(peer,),
                                device_id_type=pl.DeviceIdType.MESH)
        pl.semaphore_wait(barrier_sem, N_DEV - 1)

        xv = x_ref[:, :]
        scores = jnp.dot(xv, rw_ref[:, :], preferred_element_type=jnp.float32)
        m = jnp.max(scores, axis=1, keepdims=True)
        p = jnp.exp(scores - m)
        probs = p / jnp.sum(p, axis=1, keepdims=True)
        idx = idx_ref[:, :]
        eids = lax.broadcasted_iota(jnp.int32, (n, N_EXP), 1)
        gate_ref[:, :] = jnp.sum(jnp.where(eids == idx, probs, 0.0), axis=1,
                                 keepdims=True)

        for k in range(N_EXP_LOCAL):
            ewb_ref[k, :, :] = ew_ref[k, :, :].astype(jnp.bfloat16)
        swb_ref[:, :] = sw_ref[:, :].astype(jnp.bfloat16)

        def partial_chunk(dst):
            rows = pl.ds(dst * chunk, chunk)
            xs = x_ref[rows, :]
            idx_c = idx_ref[rows, :]
            gate_c = gate_ref[rows, :]
            acc = jnp.zeros((chunk, h), jnp.float32)
            for k in range(N_EXP_LOCAL):
                e = my * N_EXP_LOCAL + k
                w = jnp.where(idx_c == e, gate_c, 0.0)
                acc = acc + jnp.dot((xs * w).astype(jnp.bfloat16),
                                    ewb_ref[k, :, :],
                                    preferred_element_type=jnp.float32)
            return acc

        rdmas = []
        for t in range(1, N_DEV):
            dst = lax.rem(my + t, N_DEV)
            send_ref[t - 1, :, :] = partial_chunk(dst).astype(jnp.bfloat16)
            rdma = pltpu.make_async_remote_copy(
                src_ref=send_ref.at[t - 1],
                dst_ref=recv_ref.at[t - 1],
                send_sem=send_sems.at[t - 1],
                recv_sem=recv_sems.at[t - 1],
                device_id=(dst,),
                device_id_type=pl.DeviceIdType.MESH,
            )
            rdma.start()
            rdmas.append(rdma)

        xs = x_ref[pl.ds(my * chunk, chunk), :]
        total = (
            jnp.dot(xs.astype(jnp.bfloat16), swb_ref[:, :],
                    preferred_element_type=jnp.float32)
            + partial_chunk(my)
        )
        for t in range(1, N_DEV):
            rdmas[t - 1].wait_recv()
            total = total + recv_ref[t - 1, :, :].astype(jnp.float32)
        out_ref[:, :] = total

        for r in rdmas:
            r.wait_send()

    return pl.pallas_call(
        body,
        out_shape=jax.ShapeDtypeStruct((chunk, h), jnp.float32),
        in_specs=[pl.BlockSpec(memory_space=pltpu.VMEM)] * 5,
        out_specs=pl.BlockSpec(memory_space=pltpu.VMEM),
        scratch_shapes=[
            pltpu.VMEM((n, 1), jnp.float32),
            pltpu.VMEM((N_EXP_LOCAL, d, h), jnp.bfloat16),
            pltpu.VMEM((d, h), jnp.bfloat16),
            pltpu.VMEM((N_DEV - 1, chunk, h), jnp.bfloat16),
            pltpu.VMEM((N_DEV - 1, chunk, h), jnp.bfloat16),
            pltpu.SemaphoreType.DMA((N_DEV - 1,)),
            pltpu.SemaphoreType.DMA((N_DEV - 1,)),
        ],
        compiler_params=pltpu.CompilerParams(collective_id=0),
    )(x, router_W, route_idx, expert_W, shared_W)
